# Initial kernel scaffold; baseline (speedup 1.0000x reference)
import functools

import jax
import jax.numpy as jnp
from jax import lax
from jax.experimental import pallas as pl
from jax.experimental.pallas import tpu as pltpu
from jax.experimental.pallas import tpu_sc as plsc

N = 10000
E = 320000
NC = 2
NS = 16
NW = NC * NS
EPW = E // NW
CH = 80
NCH = EPW // CH
RA = 624
RB = N - (NS - 1) * RA

_MESH = plsc.VectorSubcoreMesh(core_axis_name="c", subcore_axis_name="s")


def _sc_degrees(src, dst, zeros1):
    @functools.partial(
        pl.kernel,
        out_type=(
            jax.ShapeDtypeStruct((NC, N), jnp.float32),
            jax.ShapeDtypeStruct((NC, N), jnp.float32),
        ),
        mesh=_MESH,
        scratch_types=[
            pltpu.VMEM((CH,), jnp.int32),
            pltpu.VMEM((CH,), jnp.int32),
            pltpu.VMEM((CH,), jnp.float32),
            pltpu.VMEM_SHARED((N,), jnp.float32),
            pltpu.VMEM_SHARED((N,), jnp.float32),
        ],
    )
    def k(src_hbm, dst_hbm, z_hbm, odeg_hbm, ideg_hbm, sidx, didx, ones_v, oacc, iacc):
        cid = lax.axis_index("c")
        sid = lax.axis_index("s")
        wid = sid * NC + cid

        for j in range(CH // 16):
            ones_v[pl.ds(j * 16, 16)] = jnp.ones((16,), jnp.float32)

        @pl.when(sid < NS - 1)
        def _():
            pltpu.sync_copy(z_hbm.at[pl.ds(0, RA)], oacc.at[pl.ds(sid * RA, RA)])
            pltpu.sync_copy(z_hbm.at[pl.ds(0, RA)], iacc.at[pl.ds(sid * RA, RA)])

        @pl.when(sid == NS - 1)
        def _():
            pltpu.sync_copy(z_hbm, oacc.at[pl.ds((NS - 1) * RA, RB)])
            pltpu.sync_copy(z_hbm, iacc.at[pl.ds((NS - 1) * RA, RB)])

        plsc.subcore_barrier()

        base = wid * EPW

        def body(i, carry):
            off = base + i * CH
            pltpu.sync_copy(src_hbm.at[pl.ds(off, CH)], sidx)
            pltpu.sync_copy(dst_hbm.at[pl.ds(off, CH)], didx)
            pltpu.sync_copy(ones_v, oacc.at[sidx], add=True)
            pltpu.sync_copy(ones_v, iacc.at[didx], add=True)
            return carry

        lax.fori_loop(0, NCH, body, 0)
        plsc.subcore_barrier()

        @pl.when(sid < NS - 1)
        def _():
            pltpu.sync_copy(oacc.at[pl.ds(sid * RA, RA)], odeg_hbm.at[cid, pl.ds(sid * RA, RA)])
            pltpu.sync_copy(iacc.at[pl.ds(sid * RA, RA)], ideg_hbm.at[cid, pl.ds(sid * RA, RA)])

        @pl.when(sid == NS - 1)
        def _():
            pltpu.sync_copy(oacc.at[pl.ds((NS - 1) * RA, RB)], odeg_hbm.at[cid, pl.ds((NS - 1) * RA, RB)])
            pltpu.sync_copy(iacc.at[pl.ds((NS - 1) * RA, RB)], ideg_hbm.at[cid, pl.ds((NS - 1) * RA, RB)])

    return k(src, dst, zeros1)


def _sc_edge_agg(table, src, dst, zeros2, d):
    @functools.partial(
        pl.kernel,
        out_type=jax.ShapeDtypeStruct((NC, N, d), jnp.float32),
        mesh=_MESH,
        scratch_types=[
            pltpu.VMEM((CH,), jnp.int32),
            pltpu.VMEM((CH,), jnp.int32),
            pltpu.VMEM((CH, d), jnp.float32),
            pltpu.VMEM_SHARED((N, d), jnp.float32),
            pltpu.SemaphoreType.DMA,
        ],
    )
    def k(tab_hbm, src_hbm, dst_hbm, z_hbm, out_hbm, sidx, didx, rows_v, acc, sem):
        cid = lax.axis_index("c")
        sid = lax.axis_index("s")
        wid = sid * NC + cid

        @pl.when(sid < NS - 1)
        def _():
            pltpu.sync_copy(z_hbm.at[pl.ds(0, RA)], acc.at[pl.ds(sid * RA, RA)])

        @pl.when(sid == NS - 1)
        def _():
            pltpu.sync_copy(z_hbm, acc.at[pl.ds((NS - 1) * RA, RB)])

        plsc.subcore_barrier()

        base = wid * EPW

        def body(i, carry):
            off = base + i * CH
            pltpu.sync_copy(src_hbm.at[pl.ds(off, CH)], sidx)
            pltpu.sync_copy(dst_hbm.at[pl.ds(off, CH)], didx)
            pltpu.async_copy(tab_hbm.at[sidx], rows_v, sem).wait()
            pltpu.sync_copy(rows_v, acc.at[didx], add=True)
            return carry

        lax.fori_loop(0, NCH, body, 0)
        plsc.subcore_barrier()

        @pl.when(sid < NS - 1)
        def _():
            pltpu.sync_copy(acc.at[pl.ds(sid * RA, RA)], out_hbm.at[cid, pl.ds(sid * RA, RA)])

        @pl.when(sid == NS - 1)
        def _():
            pltpu.sync_copy(acc.at[pl.ds((NS - 1) * RA, RB)], out_hbm.at[cid, pl.ds((NS - 1) * RA, RB)])

    return k(table, src, dst, zeros2)


_R = 2000


def _tc_prep(features, w1, od, idg):
    def body(x_ref, w_ref, od_ref, id_ref, hs_ref, ns_ref, nd_ref):
        ns = 1.0 / jnp.sqrt(jnp.maximum(od_ref[0] + od_ref[1], 1.0))
        nd = 1.0 / jnp.sqrt(jnp.maximum(id_ref[0] + id_ref[1], 1.0))
        ns_ref[...] = ns
        nd_ref[...] = nd
        hs_ref[...] = jnp.dot(x_ref[...] * ns, w_ref[...],
                              preferred_element_type=jnp.float32)

    return pl.pallas_call(
        body,
        grid=(N // _R,),
        in_specs=[
            pl.BlockSpec((_R, 128), lambda i: (i, 0)),
            pl.BlockSpec((128, 128), lambda i: (0, 0)),
            pl.BlockSpec((NC, _R, 1), lambda i: (0, i, 0)),
            pl.BlockSpec((NC, _R, 1), lambda i: (0, i, 0)),
        ],
        out_specs=[
            pl.BlockSpec((_R, 128), lambda i: (i, 0)),
            pl.BlockSpec((_R, 1), lambda i: (i, 0)),
            pl.BlockSpec((_R, 1), lambda i: (i, 0)),
        ],
        out_shape=[
            jax.ShapeDtypeStruct((N, 128), jnp.float32),
            jax.ShapeDtypeStruct((N, 1), jnp.float32),
            jax.ShapeDtypeStruct((N, 1), jnp.float32),
        ],
    )(features, w1, od, idg)


def _tc_mid(agg1, nd, b1, ns, w2):
    def body(p_ref, nd_ref, b_ref, ns_ref, w_ref, h1_ref, g2_ref):
        h1 = jnp.maximum((p_ref[0] + p_ref[1]) * nd_ref[...] + b_ref[...], 0.0)
        h1_ref[...] = h1
        g2_ref[...] = jnp.dot(h1 * ns_ref[...], w_ref[...],
                              preferred_element_type=jnp.float32)

    return pl.pallas_call(
        body,
        grid=(N // _R,),
        in_specs=[
            pl.BlockSpec((NC, _R, 128), lambda i: (0, i, 0)),
            pl.BlockSpec((_R, 1), lambda i: (i, 0)),
            pl.BlockSpec((1, 128), lambda i: (0, 0)),
            pl.BlockSpec((_R, 1), lambda i: (i, 0)),
            pl.BlockSpec((128, 16), lambda i: (0, 0)),
        ],
        out_specs=[
            pl.BlockSpec((_R, 128), lambda i: (i, 0)),
            pl.BlockSpec((_R, 16), lambda i: (i, 0)),
        ],
        out_shape=[
            jax.ShapeDtypeStruct((N, 128), jnp.float32),
            jax.ShapeDtypeStruct((N, 16), jnp.float32),
        ],
    )(agg1, nd, b1, ns, w2)


def _tc_out(agg2, nd, b2):
    def body(p_ref, nd_ref, b_ref, h2_ref):
        h2_ref[...] = (p_ref[0] + p_ref[1]) * nd_ref[...] + b_ref[...]

    return pl.pallas_call(
        body,
        grid=(N // _R,),
        in_specs=[
            pl.BlockSpec((NC, _R, 16), lambda i: (0, i, 0)),
            pl.BlockSpec((_R, 1), lambda i: (i, 0)),
            pl.BlockSpec((1, 16), lambda i: (0, 0)),
        ],
        out_specs=pl.BlockSpec((_R, 16), lambda i: (i, 0)),
        out_shape=jax.ShapeDtypeStruct((N, 16), jnp.float32),
    )(agg2, nd, b2)


def kernel(features, edge_index, W1, b1, W2, b2):
    src = edge_index[0]
    dst = edge_index[1]
    zeros1 = jnp.zeros((RB,), jnp.float32)
    zd1 = jnp.zeros((RB, 128), jnp.float32)
    zd2 = jnp.zeros((RB, 16), jnp.float32)

    odeg, ideg = _sc_degrees(src, dst, zeros1)
    hs1, ns, nd = _tc_prep(features, W1,
                           odeg.reshape(NC, N, 1), ideg.reshape(NC, N, 1))
    agg1 = _sc_edge_agg(hs1, src, dst, zd1, 128)
    h1, g2 = _tc_mid(agg1, nd, b1.reshape(1, 128), ns, W2)
    agg2 = _sc_edge_agg(g2, src, dst, zd2, 16)
    h2 = _tc_out(agg2, nd, b2.reshape(1, 16))
    return (h2, features, h1, h2)


# speedup vs baseline: 5.2206x; 5.2206x over previous
import functools

import jax
import jax.numpy as jnp
from jax import lax
from jax.experimental import pallas as pl
from jax.experimental.pallas import tpu as pltpu
from jax.experimental.pallas import tpu_sc as plsc

N = 10000
E = 320000
NC = 2
NS = 16
NW = NC * NS
EPW = E // NW
CH = 80
NCH = EPW // CH
RA = 624
RB = N - (NS - 1) * RA
W = 104

_MESH = plsc.VectorSubcoreMesh(core_axis_name="c", subcore_axis_name="s")


def _sc_degrees(src, dst, zeros1):
    @functools.partial(
        pl.kernel,
        out_type=(
            jax.ShapeDtypeStruct((NC * N,), jnp.float32),
            jax.ShapeDtypeStruct((NC * N,), jnp.float32),
        ),
        mesh=_MESH,
        scratch_types=[
            pltpu.VMEM((CH,), jnp.int32),
            pltpu.VMEM((CH,), jnp.int32),
            pltpu.VMEM((CH,), jnp.float32),
            pltpu.VMEM((RB,), jnp.float32),
            pltpu.VMEM_SHARED((N,), jnp.float32),
            pltpu.VMEM_SHARED((N,), jnp.float32),
        ],
    )
    def k(src_hbm, dst_hbm, z_hbm, odeg_hbm, ideg_hbm, sidx, didx, ones_v, zv, oacc, iacc):
        cid = lax.axis_index("c")
        sid = lax.axis_index("s")
        wid = sid * NC + cid

        for j in range(CH // 16):
            ones_v[pl.ds(j * 16, 16)] = jnp.ones((16,), jnp.float32)

        pltpu.sync_copy(z_hbm, zv)

        @pl.when(sid < NS - 1)
        def _():
            pltpu.sync_copy(zv.at[pl.ds(0, RA)], oacc.at[pl.ds(sid * RA, RA)])
            pltpu.sync_copy(zv.at[pl.ds(0, RA)], iacc.at[pl.ds(sid * RA, RA)])

        @pl.when(sid == NS - 1)
        def _():
            pltpu.sync_copy(zv, oacc.at[pl.ds((NS - 1) * RA, RB)])
            pltpu.sync_copy(zv, iacc.at[pl.ds((NS - 1) * RA, RB)])

        plsc.subcore_barrier()

        base = wid * EPW

        def body(i, carry):
            off = base + i * CH
            pltpu.sync_copy(src_hbm.at[pl.ds(off, CH)], sidx)
            pltpu.sync_copy(dst_hbm.at[pl.ds(off, CH)], didx)
            pltpu.sync_copy(ones_v, oacc.at[sidx], add=True)
            pltpu.sync_copy(ones_v, iacc.at[didx], add=True)
            return carry

        lax.fori_loop(0, NCH, body, 0)
        plsc.subcore_barrier()

        @pl.when(sid < NS - 1)
        def _():
            pltpu.sync_copy(oacc.at[pl.ds(sid * RA, RA)], zv.at[pl.ds(0, RA)])
            pltpu.sync_copy(zv.at[pl.ds(0, RA)], odeg_hbm.at[pl.ds(cid * N + sid * RA, RA)])
            pltpu.sync_copy(iacc.at[pl.ds(sid * RA, RA)], zv.at[pl.ds(0, RA)])
            pltpu.sync_copy(zv.at[pl.ds(0, RA)], ideg_hbm.at[pl.ds(cid * N + sid * RA, RA)])

        @pl.when(sid == NS - 1)
        def _():
            pltpu.sync_copy(oacc.at[pl.ds((NS - 1) * RA, RB)], zv)
            pltpu.sync_copy(zv, odeg_hbm.at[pl.ds(cid * N + (NS - 1) * RA, RB)])
            pltpu.sync_copy(iacc.at[pl.ds((NS - 1) * RA, RB)], zv)
            pltpu.sync_copy(zv, ideg_hbm.at[pl.ds(cid * N + (NS - 1) * RA, RB)])

    return k(src, dst, zeros1)


def _sc_edge_agg(table, src, dst, zeros2, d):
    @functools.partial(
        pl.kernel,
        out_type=jax.ShapeDtypeStruct((NC * N, d), jnp.float32),
        mesh=_MESH,
        scratch_types=[
            pltpu.VMEM((CH,), jnp.int32),
            pltpu.VMEM((CH,), jnp.int32),
            pltpu.VMEM((CH, d), jnp.float32),
            pltpu.VMEM((W, d), jnp.float32),
            pltpu.VMEM_SHARED((N, d), jnp.float32),
            pltpu.SemaphoreType.DMA,
        ],
        compiler_params=pltpu.CompilerParams(use_tc_tiling_on_sc=(d == 128)),
    )
    def k(tab_hbm, src_hbm, dst_hbm, z_hbm, out_hbm, sidx, didx, rows_v, zv, acc, sem):
        cid = lax.axis_index("c")
        sid = lax.axis_index("s")
        wid = sid * NC + cid

        pltpu.sync_copy(z_hbm, zv)
        for j in range(RA // W):
            pltpu.sync_copy(zv, acc.at[pl.ds(sid * RA + j * W, W)])

        @pl.when(sid == NS - 1)
        def _():
            pltpu.sync_copy(zv.at[pl.ds(0, RB - RA)], acc.at[pl.ds(N - (RB - RA), RB - RA)])

        plsc.subcore_barrier()

        base = wid * EPW

        def body(i, carry):
            off = base + i * CH
            pltpu.sync_copy(src_hbm.at[pl.ds(off, CH)], sidx)
            pltpu.sync_copy(dst_hbm.at[pl.ds(off, CH)], didx)
            pltpu.async_copy(tab_hbm.at[sidx], rows_v, sem).wait()
            pltpu.sync_copy(rows_v, acc.at[didx], add=True)
            return carry

        lax.fori_loop(0, NCH, body, 0)
        plsc.subcore_barrier()

        for j in range(RA // W):
            pltpu.sync_copy(acc.at[pl.ds(sid * RA + j * W, W)], zv)
            pltpu.sync_copy(zv, out_hbm.at[pl.ds(cid * N + sid * RA + j * W, W)])

        @pl.when(sid == NS - 1)
        def _():
            pltpu.sync_copy(acc.at[pl.ds(N - (RB - RA), RB - RA)], zv.at[pl.ds(0, RB - RA)])
            pltpu.sync_copy(zv.at[pl.ds(0, RB - RA)], out_hbm.at[pl.ds(cid * N + N - (RB - RA), RB - RA)])

    return k(table, src, dst, zeros2)


_R = 2000


def _tc_prep(features, w1, od, idg):
    def body(x_ref, w_ref, od_ref, id_ref, hs_ref, ns_ref, nd_ref):
        ns = 1.0 / jnp.sqrt(jnp.maximum(od_ref[0] + od_ref[1], 1.0))
        nd = 1.0 / jnp.sqrt(jnp.maximum(id_ref[0] + id_ref[1], 1.0))
        ns_ref[...] = ns
        nd_ref[...] = nd
        hs_ref[...] = jnp.dot(x_ref[...] * ns, w_ref[...],
                              preferred_element_type=jnp.float32)

    return pl.pallas_call(
        body,
        grid=(N // _R,),
        in_specs=[
            pl.BlockSpec((_R, 128), lambda i: (i, 0)),
            pl.BlockSpec((128, 128), lambda i: (0, 0)),
            pl.BlockSpec((NC, _R, 1), lambda i: (0, i, 0)),
            pl.BlockSpec((NC, _R, 1), lambda i: (0, i, 0)),
        ],
        out_specs=[
            pl.BlockSpec((_R, 128), lambda i: (i, 0)),
            pl.BlockSpec((_R, 1), lambda i: (i, 0)),
            pl.BlockSpec((_R, 1), lambda i: (i, 0)),
        ],
        out_shape=[
            jax.ShapeDtypeStruct((N, 128), jnp.float32),
            jax.ShapeDtypeStruct((N, 1), jnp.float32),
            jax.ShapeDtypeStruct((N, 1), jnp.float32),
        ],
    )(features, w1, od, idg)


def _tc_mid(agg1, nd, b1, ns, w2):
    def body(p_ref, nd_ref, b_ref, ns_ref, w_ref, h1_ref, g2_ref):
        h1 = jnp.maximum((p_ref[0] + p_ref[1]) * nd_ref[...] + b_ref[...], 0.0)
        h1_ref[...] = h1
        g2_ref[...] = jnp.dot(h1 * ns_ref[...], w_ref[...],
                              preferred_element_type=jnp.float32)

    return pl.pallas_call(
        body,
        grid=(N // _R,),
        in_specs=[
            pl.BlockSpec((NC, _R, 128), lambda i: (0, i, 0)),
            pl.BlockSpec((_R, 1), lambda i: (i, 0)),
            pl.BlockSpec((1, 128), lambda i: (0, 0)),
            pl.BlockSpec((_R, 1), lambda i: (i, 0)),
            pl.BlockSpec((128, 16), lambda i: (0, 0)),
        ],
        out_specs=[
            pl.BlockSpec((_R, 128), lambda i: (i, 0)),
            pl.BlockSpec((_R, 16), lambda i: (i, 0)),
        ],
        out_shape=[
            jax.ShapeDtypeStruct((N, 128), jnp.float32),
            jax.ShapeDtypeStruct((N, 16), jnp.float32),
        ],
    )(agg1, nd, b1, ns, w2)


def _tc_out(agg2, nd, b2):
    def body(p_ref, nd_ref, b_ref, h2_ref):
        h2_ref[...] = (p_ref[0] + p_ref[1]) * nd_ref[...] + b_ref[...]

    return pl.pallas_call(
        body,
        grid=(N // _R,),
        in_specs=[
            pl.BlockSpec((NC, _R, 16), lambda i: (0, i, 0)),
            pl.BlockSpec((_R, 1), lambda i: (i, 0)),
            pl.BlockSpec((1, 16), lambda i: (0, 0)),
        ],
        out_specs=pl.BlockSpec((_R, 16), lambda i: (i, 0)),
        out_shape=jax.ShapeDtypeStruct((N, 16), jnp.float32),
    )(agg2, nd, b2)


def kernel(features, edge_index, W1, b1, W2, b2):
    src = edge_index[0]
    dst = edge_index[1]
    zeros1 = jnp.zeros((RB,), jnp.float32)
    zd1 = jnp.zeros((W, 128), jnp.float32)
    zd2 = jnp.zeros((W, 16), jnp.float32)

    odeg, ideg = _sc_degrees(src, dst, zeros1)
    hs1, ns, nd = _tc_prep(features, W1,
                           odeg.reshape(NC, N, 1), ideg.reshape(NC, N, 1))
    agg1 = _sc_edge_agg(hs1, src, dst, zd1, 128).reshape(NC, N, 128)
    h1, g2 = _tc_mid(agg1, nd, b1.reshape(1, 128), ns, W2)
    agg2 = _sc_edge_agg(g2, src, dst, zd2, 16).reshape(NC, N, 16)
    h2 = _tc_out(agg2, nd, b2.reshape(1, 16))
    return (h2, features, h1, h2)


# speedup vs baseline: 11.6613x; 2.2337x over previous
import functools

import jax
import jax.numpy as jnp
from jax import lax
from jax.experimental import pallas as pl
from jax.experimental.pallas import tpu as pltpu
from jax.experimental.pallas import tpu_sc as plsc

N = 10000
E = 320000
NC = 2
NS = 16
NW = NC * NS
CH = 128
NROW = E // CH
CPW = NROW // NW
BLK = 6
NBLK = CPW // BLK
XTRA = NROW - NW * CPW
RA = 624
RB = N - (NS - 1) * RA
W = 48

_MESH = plsc.VectorSubcoreMesh(core_axis_name="c", subcore_axis_name="s")


def _sc_degrees(src2d, dst2d, zeros1):
    @functools.partial(
        pl.kernel,
        out_type=(
            jax.ShapeDtypeStruct((NC * N,), jnp.float32),
            jax.ShapeDtypeStruct((NC * N,), jnp.float32),
        ),
        mesh=_MESH,
        scratch_types=[
            pltpu.VMEM((BLK, CH), jnp.int32),
            pltpu.VMEM((BLK, CH), jnp.int32),
            pltpu.VMEM((CH,), jnp.float32),
            pltpu.VMEM((RB,), jnp.float32),
            pltpu.VMEM_SHARED((N,), jnp.float32),
            pltpu.VMEM_SHARED((N,), jnp.float32),
            pltpu.SemaphoreType.DMA,
            pltpu.SemaphoreType.DMA,
        ],
        compiler_params=pltpu.CompilerParams(use_tc_tiling_on_sc=False),
    )
    def k(src_hbm, dst_hbm, z_hbm, odeg_hbm, ideg_hbm,
          sidx, didx, ones_v, zv, oacc, iacc, semA, semB):
        cid = lax.axis_index("c")
        sid = lax.axis_index("s")
        wid = sid * NC + cid

        for j in range(CH // 16):
            ones_v[pl.ds(j * 16, 16)] = jnp.ones((16,), jnp.float32)

        pltpu.sync_copy(z_hbm, zv)

        @pl.when(sid < NS - 1)
        def _():
            pltpu.sync_copy(zv.at[pl.ds(0, RA)], oacc.at[pl.ds(sid * RA, RA)])
            pltpu.sync_copy(zv.at[pl.ds(0, RA)], iacc.at[pl.ds(sid * RA, RA)])

        @pl.when(sid == NS - 1)
        def _():
            pltpu.sync_copy(zv, oacc.at[pl.ds((NS - 1) * RA, RB)])
            pltpu.sync_copy(zv, iacc.at[pl.ds((NS - 1) * RA, RB)])

        plsc.subcore_barrier()

        c0 = wid * CPW

        def body(b, carry):
            @pl.when(b > 0)
            def _():
                pltpu.make_async_copy(ones_v, oacc.at[sidx.at[BLK - 1]], semA).wait()
                pltpu.make_async_copy(ones_v, iacc.at[didx.at[BLK - 1]], semB).wait()

            pltpu.sync_copy(src_hbm.at[pl.ds(c0 + b * BLK, BLK)], sidx)
            pltpu.sync_copy(dst_hbm.at[pl.ds(c0 + b * BLK, BLK)], didx)
            descs = {}
            for kk in range(BLK):
                if kk >= 2:
                    descs[kk - 2][0].wait()
                    descs[kk - 2][1].wait()
                descs[kk] = (
                    pltpu.async_copy(ones_v, oacc.at[sidx.at[kk]], semA, add=True),
                    pltpu.async_copy(ones_v, iacc.at[didx.at[kk]], semB, add=True),
                )
            descs[BLK - 2][0].wait()
            descs[BLK - 2][1].wait()
            return carry

        lax.fori_loop(0, NBLK, body, 0)
        pltpu.make_async_copy(ones_v, oacc.at[sidx.at[BLK - 1]], semA).wait()
        pltpu.make_async_copy(ones_v, iacc.at[didx.at[BLK - 1]], semB).wait()

        @pl.when(wid < XTRA)
        def _():
            pltpu.sync_copy(src_hbm.at[pl.ds(NW * CPW + wid, 1)], sidx.at[pl.ds(0, 1)])
            pltpu.sync_copy(dst_hbm.at[pl.ds(NW * CPW + wid, 1)], didx.at[pl.ds(0, 1)])
            pltpu.sync_copy(ones_v, oacc.at[sidx.at[0]], add=True)
            pltpu.sync_copy(ones_v, iacc.at[didx.at[0]], add=True)

        plsc.subcore_barrier()

        @pl.when(sid < NS - 1)
        def _():
            pltpu.sync_copy(oacc.at[pl.ds(sid * RA, RA)], zv.at[pl.ds(0, RA)])
            pltpu.sync_copy(zv.at[pl.ds(0, RA)], odeg_hbm.at[pl.ds(cid * N + sid * RA, RA)])
            pltpu.sync_copy(iacc.at[pl.ds(sid * RA, RA)], zv.at[pl.ds(0, RA)])
            pltpu.sync_copy(zv.at[pl.ds(0, RA)], ideg_hbm.at[pl.ds(cid * N + sid * RA, RA)])

        @pl.when(sid == NS - 1)
        def _():
            pltpu.sync_copy(oacc.at[pl.ds((NS - 1) * RA, RB)], zv)
            pltpu.sync_copy(zv, odeg_hbm.at[pl.ds(cid * N + (NS - 1) * RA, RB)])
            pltpu.sync_copy(iacc.at[pl.ds((NS - 1) * RA, RB)], zv)
            pltpu.sync_copy(zv, ideg_hbm.at[pl.ds(cid * N + (NS - 1) * RA, RB)])

    return k(src2d, dst2d, zeros1)


def _sc_edge_agg(table, src2d, dst2d, zeros2, d):
    @functools.partial(
        pl.kernel,
        out_type=jax.ShapeDtypeStruct((NC * N, d), jnp.float32),
        mesh=_MESH,
        scratch_types=[
            pltpu.VMEM((BLK, CH), jnp.int32),
            pltpu.VMEM((BLK, CH), jnp.int32),
            pltpu.VMEM((CH, d), jnp.float32),
            pltpu.VMEM((CH, d), jnp.float32),
            pltpu.VMEM((W, d), jnp.float32),
            pltpu.VMEM_SHARED((N, d), jnp.float32),
            pltpu.SemaphoreType.DMA,
            pltpu.SemaphoreType.DMA,
            pltpu.SemaphoreType.DMA,
            pltpu.SemaphoreType.DMA,
        ],
        compiler_params=pltpu.CompilerParams(use_tc_tiling_on_sc=False),
    )
    def k(tab_hbm, src_hbm, dst_hbm, z_hbm, out_hbm,
          sidx, didx, rowsA, rowsB, zv, acc, semGA, semGB, semSA, semSB):
        cid = lax.axis_index("c")
        sid = lax.axis_index("s")
        wid = sid * NC + cid

        pltpu.sync_copy(z_hbm, zv)
        for j in range(RA // W):
            pltpu.sync_copy(zv, acc.at[pl.ds(sid * RA + j * W, W)])

        @pl.when(sid == NS - 1)
        def _():
            pltpu.sync_copy(zv.at[pl.ds(0, RB - RA)], acc.at[pl.ds(N - (RB - RA), RB - RA)])

        plsc.subcore_barrier()

        c0 = wid * CPW
        rows = [rowsA, rowsB]
        semG = [semGA, semGB]
        semS = [semSA, semSB]

        def body(b, carry):
            @pl.when(b > 0)
            def _():
                pltpu.make_async_copy(rowsB, acc.at[didx.at[BLK - 1]], semSB).wait()

            pltpu.sync_copy(src_hbm.at[pl.ds(c0 + b * BLK, BLK)], sidx)
            pltpu.sync_copy(dst_hbm.at[pl.ds(c0 + b * BLK, BLK)], didx)
            descG = {0: pltpu.async_copy(tab_hbm.at[sidx.at[0]], rowsA, semGA)}
            descS = {}
            for kk in range(BLK):
                if kk < BLK - 1:
                    if kk >= 1:
                        descS[kk - 1].wait()
                    descG[kk + 1] = pltpu.async_copy(
                        tab_hbm.at[sidx.at[kk + 1]], rows[(kk + 1) % 2], semG[(kk + 1) % 2])
                descG[kk].wait()
                descS[kk] = pltpu.async_copy(
                    rows[kk % 2], acc.at[didx.at[kk]], semS[kk % 2], add=True)
            descS[BLK - 2].wait()
            return carry

        lax.fori_loop(0, NBLK, body, 0)
        pltpu.make_async_copy(rowsB, acc.at[didx.at[BLK - 1]], semSB).wait()

        @pl.when(wid < XTRA)
        def _():
            pltpu.sync_copy(src_hbm.at[pl.ds(NW * CPW + wid, 1)], sidx.at[pl.ds(0, 1)])
            pltpu.sync_copy(dst_hbm.at[pl.ds(NW * CPW + wid, 1)], didx.at[pl.ds(0, 1)])
            pltpu.async_copy(tab_hbm.at[sidx.at[0]], rowsA, semGA).wait()
            pltpu.sync_copy(rowsA, acc.at[didx.at[0]], add=True)

        plsc.subcore_barrier()

        for j in range(RA // W):
            pltpu.sync_copy(acc.at[pl.ds(sid * RA + j * W, W)], zv)
            pltpu.sync_copy(zv, out_hbm.at[pl.ds(cid * N + sid * RA + j * W, W)])

        @pl.when(sid == NS - 1)
        def _():
            pltpu.sync_copy(acc.at[pl.ds(N - (RB - RA), RB - RA)], zv.at[pl.ds(0, RB - RA)])
            pltpu.sync_copy(zv.at[pl.ds(0, RB - RA)], out_hbm.at[pl.ds(cid * N + N - (RB - RA), RB - RA)])

    return k(table, src2d, dst2d, zeros2)


_R = 2000


def _tc_prep(features, w1, od, idg):
    def body(x_ref, w_ref, od_ref, id_ref, hs_ref, ns_ref, nd_ref):
        ns = 1.0 / jnp.sqrt(jnp.maximum(od_ref[0] + od_ref[1], 1.0))
        nd = 1.0 / jnp.sqrt(jnp.maximum(id_ref[0] + id_ref[1], 1.0))
        ns_ref[...] = ns
        nd_ref[...] = nd
        hs_ref[...] = jnp.dot(x_ref[...] * ns, w_ref[...],
                              preferred_element_type=jnp.float32)

    return pl.pallas_call(
        body,
        grid=(N // _R,),
        in_specs=[
            pl.BlockSpec((_R, 128), lambda i: (i, 0)),
            pl.BlockSpec((128, 128), lambda i: (0, 0)),
            pl.BlockSpec((NC, _R, 1), lambda i: (0, i, 0)),
            pl.BlockSpec((NC, _R, 1), lambda i: (0, i, 0)),
        ],
        out_specs=[
            pl.BlockSpec((_R, 128), lambda i: (i, 0)),
            pl.BlockSpec((_R, 1), lambda i: (i, 0)),
            pl.BlockSpec((_R, 1), lambda i: (i, 0)),
        ],
        out_shape=[
            jax.ShapeDtypeStruct((N, 128), jnp.float32),
            jax.ShapeDtypeStruct((N, 1), jnp.float32),
            jax.ShapeDtypeStruct((N, 1), jnp.float32),
        ],
    )(features, w1, od, idg)


def _tc_mid(agg1, nd, b1, ns, w2):
    def body(p_ref, nd_ref, b_ref, ns_ref, w_ref, h1_ref, g2_ref):
        h1 = jnp.maximum((p_ref[0] + p_ref[1]) * nd_ref[...] + b_ref[...], 0.0)
        h1_ref[...] = h1
        g2_ref[...] = jnp.dot(h1 * ns_ref[...], w_ref[...],
                              preferred_element_type=jnp.float32)

    return pl.pallas_call(
        body,
        grid=(N // _R,),
        in_specs=[
            pl.BlockSpec((NC, _R, 128), lambda i: (0, i, 0)),
            pl.BlockSpec((_R, 1), lambda i: (i, 0)),
            pl.BlockSpec((1, 128), lambda i: (0, 0)),
            pl.BlockSpec((_R, 1), lambda i: (i, 0)),
            pl.BlockSpec((128, 16), lambda i: (0, 0)),
        ],
        out_specs=[
            pl.BlockSpec((_R, 128), lambda i: (i, 0)),
            pl.BlockSpec((_R, 16), lambda i: (i, 0)),
        ],
        out_shape=[
            jax.ShapeDtypeStruct((N, 128), jnp.float32),
            jax.ShapeDtypeStruct((N, 16), jnp.float32),
        ],
    )(agg1, nd, b1, ns, w2)


def _tc_out(agg2, nd, b2):
    def body(p_ref, nd_ref, b_ref, h2_ref):
        h2_ref[...] = (p_ref[0] + p_ref[1]) * nd_ref[...] + b_ref[...]

    return pl.pallas_call(
        body,
        grid=(N // _R,),
        in_specs=[
            pl.BlockSpec((NC, _R, 16), lambda i: (0, i, 0)),
            pl.BlockSpec((_R, 1), lambda i: (i, 0)),
            pl.BlockSpec((1, 16), lambda i: (0, 0)),
        ],
        out_specs=pl.BlockSpec((_R, 16), lambda i: (i, 0)),
        out_shape=jax.ShapeDtypeStruct((N, 16), jnp.float32),
    )(agg2, nd, b2)


def kernel(features, edge_index, W1, b1, W2, b2):
    src2d = edge_index[0].reshape(NROW, CH)
    dst2d = edge_index[1].reshape(NROW, CH)
    zeros1 = jnp.zeros((RB,), jnp.float32)
    zd1 = jnp.zeros((W, 128), jnp.float32)
    zd2 = jnp.zeros((W, 16), jnp.float32)

    odeg, ideg = _sc_degrees(src2d, dst2d, zeros1)
    hs1, ns, nd = _tc_prep(features, W1,
                           odeg.reshape(NC, N, 1), ideg.reshape(NC, N, 1))
    agg1 = _sc_edge_agg(hs1, src2d, dst2d, zd1, 128).reshape(NC, N, 128)
    h1, g2 = _tc_mid(agg1, nd, b1.reshape(1, 128), ns, W2)
    agg2 = _sc_edge_agg(g2, src2d, dst2d, zd2, 16).reshape(NC, N, 16)
    h2 = _tc_out(agg2, nd, b2.reshape(1, 16))
    return (h2, features, h1, h2)


# speedup vs baseline: 11.9374x; 1.0237x over previous
import functools

import jax
import jax.numpy as jnp
from jax import lax
from jax.experimental import pallas as pl
from jax.experimental.pallas import tpu as pltpu
from jax.experimental.pallas import tpu_sc as plsc

N = 10000
E = 320000
NC = 2
NS = 16
NW = NC * NS
CH = 128
NROW = E // CH
CPW = NROW // NW
BLK = 6
NBLK = CPW // BLK
XTRA = NROW - NW * CPW
RA = 624
RB = N - (NS - 1) * RA
W = 48

_MESH = plsc.VectorSubcoreMesh(core_axis_name="c", subcore_axis_name="s")


def _sc_degrees(src2d, dst2d, zeros1):
    @functools.partial(
        pl.kernel,
        out_type=(
            jax.ShapeDtypeStruct((NC * N,), jnp.float32),
            jax.ShapeDtypeStruct((NC * N,), jnp.float32),
        ),
        mesh=_MESH,
        scratch_types=[
            pltpu.VMEM((BLK, CH), jnp.int32),
            pltpu.VMEM((BLK, CH), jnp.int32),
            pltpu.VMEM((CH,), jnp.float32),
            pltpu.VMEM((RB,), jnp.float32),
            pltpu.VMEM_SHARED((N,), jnp.float32),
            pltpu.VMEM_SHARED((N,), jnp.float32),
            pltpu.SemaphoreType.DMA,
            pltpu.SemaphoreType.DMA,
        ],
        compiler_params=pltpu.CompilerParams(use_tc_tiling_on_sc=False),
    )
    def k(src_hbm, dst_hbm, z_hbm, odeg_hbm, ideg_hbm,
          sidx, didx, ones_v, zv, oacc, iacc, semA, semB):
        cid = lax.axis_index("c")
        sid = lax.axis_index("s")
        wid = sid * NC + cid

        for j in range(CH // 16):
            ones_v[pl.ds(j * 16, 16)] = jnp.ones((16,), jnp.float32)

        pltpu.sync_copy(z_hbm, zv)

        @pl.when(sid < NS - 1)
        def _():
            pltpu.sync_copy(zv.at[pl.ds(0, RA)], oacc.at[pl.ds(sid * RA, RA)])
            pltpu.sync_copy(zv.at[pl.ds(0, RA)], iacc.at[pl.ds(sid * RA, RA)])

        @pl.when(sid == NS - 1)
        def _():
            pltpu.sync_copy(zv, oacc.at[pl.ds((NS - 1) * RA, RB)])
            pltpu.sync_copy(zv, iacc.at[pl.ds((NS - 1) * RA, RB)])

        plsc.subcore_barrier()

        c0 = wid * CPW

        def body(b, carry):
            @pl.when(b > 0)
            def _():
                pltpu.make_async_copy(ones_v, oacc.at[sidx.at[BLK - 1]], semA).wait()
                pltpu.make_async_copy(ones_v, iacc.at[didx.at[BLK - 1]], semB).wait()

            pltpu.sync_copy(src_hbm.at[pl.ds(c0 + b * BLK, BLK)], sidx)
            pltpu.sync_copy(dst_hbm.at[pl.ds(c0 + b * BLK, BLK)], didx)
            descs = {}
            for kk in range(BLK):
                if kk >= 1:
                    descs[kk - 1][0].wait()
                    descs[kk - 1][1].wait()
                descs[kk] = (
                    pltpu.async_copy(ones_v, oacc.at[sidx.at[kk]], semA, add=True),
                    pltpu.async_copy(ones_v, iacc.at[didx.at[kk]], semB, add=True),
                )
            return carry

        lax.fori_loop(0, NBLK, body, 0)
        pltpu.make_async_copy(ones_v, oacc.at[sidx.at[BLK - 1]], semA).wait()
        pltpu.make_async_copy(ones_v, iacc.at[didx.at[BLK - 1]], semB).wait()

        @pl.when(wid < XTRA)
        def _():
            pltpu.sync_copy(src_hbm.at[pl.ds(NW * CPW + wid, 1)], sidx.at[pl.ds(0, 1)])
            pltpu.sync_copy(dst_hbm.at[pl.ds(NW * CPW + wid, 1)], didx.at[pl.ds(0, 1)])
            pltpu.sync_copy(ones_v, oacc.at[sidx.at[0]], add=True)
            pltpu.sync_copy(ones_v, iacc.at[didx.at[0]], add=True)

        plsc.subcore_barrier()

        @pl.when(sid < NS - 1)
        def _():
            pltpu.sync_copy(oacc.at[pl.ds(sid * RA, RA)], zv.at[pl.ds(0, RA)])
            pltpu.sync_copy(zv.at[pl.ds(0, RA)], odeg_hbm.at[pl.ds(cid * N + sid * RA, RA)])
            pltpu.sync_copy(iacc.at[pl.ds(sid * RA, RA)], zv.at[pl.ds(0, RA)])
            pltpu.sync_copy(zv.at[pl.ds(0, RA)], ideg_hbm.at[pl.ds(cid * N + sid * RA, RA)])

        @pl.when(sid == NS - 1)
        def _():
            pltpu.sync_copy(oacc.at[pl.ds((NS - 1) * RA, RB)], zv)
            pltpu.sync_copy(zv, odeg_hbm.at[pl.ds(cid * N + (NS - 1) * RA, RB)])
            pltpu.sync_copy(iacc.at[pl.ds((NS - 1) * RA, RB)], zv)
            pltpu.sync_copy(zv, ideg_hbm.at[pl.ds(cid * N + (NS - 1) * RA, RB)])

    return k(src2d, dst2d, zeros1)


def _sc_edge_agg(table, src2d, dst2d, zeros2, d):
    @functools.partial(
        pl.kernel,
        out_type=jax.ShapeDtypeStruct((NC * N, d), jnp.float32),
        mesh=_MESH,
        scratch_types=[
            pltpu.VMEM((BLK, CH), jnp.int32),
            pltpu.VMEM((BLK, CH), jnp.int32),
            pltpu.VMEM((CH, d), jnp.float32),
            pltpu.VMEM((CH, d), jnp.float32),
            pltpu.VMEM((CH, d), jnp.float32),
            pltpu.VMEM_SHARED((N, d), jnp.float32),
            pltpu.SemaphoreType.DMA,
            pltpu.SemaphoreType.DMA,
            pltpu.SemaphoreType.DMA,
            pltpu.SemaphoreType.DMA,
            pltpu.SemaphoreType.DMA,
            pltpu.SemaphoreType.DMA,
        ],
        compiler_params=pltpu.CompilerParams(use_tc_tiling_on_sc=False),
    )
    def k(tab_hbm, src_hbm, dst_hbm, z_hbm, out_hbm,
          sidx, didx, rows0, rows1, rows2, acc,
          semG0, semG1, semG2, semS0, semS1, semS2):
        cid = lax.axis_index("c")
        sid = lax.axis_index("s")
        wid = sid * NC + cid
        rows = [rows0, rows1, rows2]
        semG = [semG0, semG1, semG2]
        semS = [semS0, semS1, semS2]

        pltpu.sync_copy(z_hbm, rows0)

        @pl.when(sid < NS - 1)
        def _():
            for j in range(4):
                pltpu.sync_copy(rows0, acc.at[pl.ds(sid * RA + j * CH, CH)])
            pltpu.sync_copy(rows0.at[pl.ds(0, RA - 4 * CH)],
                            acc.at[pl.ds(sid * RA + 4 * CH, RA - 4 * CH)])

        @pl.when(sid == NS - 1)
        def _():
            for j in range(5):
                pltpu.sync_copy(rows0, acc.at[pl.ds((NS - 1) * RA + j * CH, CH)])

        plsc.subcore_barrier()

        c0 = wid * CPW

        def body(b, carry):
            @pl.when(b > 0)
            def _():
                pltpu.make_async_copy(
                    rows[(BLK - 1) % 3], acc.at[didx.at[BLK - 1]], semS[(BLK - 1) % 3]).wait()

            pltpu.sync_copy(src_hbm.at[pl.ds(c0 + b * BLK, BLK)], sidx)
            pltpu.sync_copy(dst_hbm.at[pl.ds(c0 + b * BLK, BLK)], didx)
            descG = {}
            descS = {}
            for g in range(2):
                descG[g] = pltpu.async_copy(tab_hbm.at[sidx.at[g]], rows[g], semG[g])
            for kk in range(BLK):
                if kk >= 1:
                    descS[kk - 1].wait()
                if kk + 2 < BLK:
                    descG[kk + 2] = pltpu.async_copy(
                        tab_hbm.at[sidx.at[kk + 2]], rows[(kk + 2) % 3], semG[(kk + 2) % 3])
                descG[kk].wait()
                descS[kk] = pltpu.async_copy(
                    rows[kk % 3], acc.at[didx.at[kk]], semS[kk % 3], add=True)
            return carry

        lax.fori_loop(0, NBLK, body, 0)
        pltpu.make_async_copy(
            rows[(BLK - 1) % 3], acc.at[didx.at[BLK - 1]], semS[(BLK - 1) % 3]).wait()

        @pl.when(wid < XTRA)
        def _():
            pltpu.sync_copy(src_hbm.at[pl.ds(NW * CPW + wid, 1)], sidx.at[pl.ds(0, 1)])
            pltpu.sync_copy(dst_hbm.at[pl.ds(NW * CPW + wid, 1)], didx.at[pl.ds(0, 1)])
            pltpu.async_copy(tab_hbm.at[sidx.at[0]], rows0, semG0).wait()
            pltpu.sync_copy(rows0, acc.at[didx.at[0]], add=True)

        plsc.subcore_barrier()

        @pl.when(sid < NS - 1)
        def _():
            for j in range(4):
                pltpu.sync_copy(acc.at[pl.ds(sid * RA + j * CH, CH)], rows0)
                pltpu.sync_copy(rows0, out_hbm.at[pl.ds(cid * N + sid * RA + j * CH, CH)])
            pltpu.sync_copy(acc.at[pl.ds(sid * RA + 4 * CH, RA - 4 * CH)],
                            rows0.at[pl.ds(0, RA - 4 * CH)])
            pltpu.sync_copy(rows0.at[pl.ds(0, RA - 4 * CH)],
                            out_hbm.at[pl.ds(cid * N + sid * RA + 4 * CH, RA - 4 * CH)])

        @pl.when(sid == NS - 1)
        def _():
            for j in range(5):
                pltpu.sync_copy(acc.at[pl.ds((NS - 1) * RA + j * CH, CH)], rows0)
                pltpu.sync_copy(rows0, out_hbm.at[pl.ds(cid * N + (NS - 1) * RA + j * CH, CH)])

    return k(table, src2d, dst2d, zeros2)


_R = 2000


def _tc_prep(features, w1, od, idg):
    def body(x_ref, w_ref, od_ref, id_ref, hs_ref, ns_ref, nd_ref):
        ns = 1.0 / jnp.sqrt(jnp.maximum(od_ref[0] + od_ref[1], 1.0))
        nd = 1.0 / jnp.sqrt(jnp.maximum(id_ref[0] + id_ref[1], 1.0))
        ns_ref[...] = ns
        nd_ref[...] = nd
        hs_ref[...] = jnp.dot(x_ref[...] * ns, w_ref[...],
                              preferred_element_type=jnp.float32)

    return pl.pallas_call(
        body,
        grid=(N // _R,),
        in_specs=[
            pl.BlockSpec((_R, 128), lambda i: (i, 0)),
            pl.BlockSpec((128, 128), lambda i: (0, 0)),
            pl.BlockSpec((NC, _R, 1), lambda i: (0, i, 0)),
            pl.BlockSpec((NC, _R, 1), lambda i: (0, i, 0)),
        ],
        out_specs=[
            pl.BlockSpec((_R, 128), lambda i: (i, 0)),
            pl.BlockSpec((_R, 1), lambda i: (i, 0)),
            pl.BlockSpec((_R, 1), lambda i: (i, 0)),
        ],
        out_shape=[
            jax.ShapeDtypeStruct((N, 128), jnp.float32),
            jax.ShapeDtypeStruct((N, 1), jnp.float32),
            jax.ShapeDtypeStruct((N, 1), jnp.float32),
        ],
    )(features, w1, od, idg)


def _tc_mid(agg1, nd, b1, ns, w2):
    def body(p_ref, nd_ref, b_ref, ns_ref, w_ref, h1_ref, g2_ref):
        h1 = jnp.maximum((p_ref[0] + p_ref[1]) * nd_ref[...] + b_ref[...], 0.0)
        h1_ref[...] = h1
        g2_ref[...] = jnp.dot(h1 * ns_ref[...], w_ref[...],
                              preferred_element_type=jnp.float32)

    return pl.pallas_call(
        body,
        grid=(N // _R,),
        in_specs=[
            pl.BlockSpec((NC, _R, 128), lambda i: (0, i, 0)),
            pl.BlockSpec((_R, 1), lambda i: (i, 0)),
            pl.BlockSpec((1, 128), lambda i: (0, 0)),
            pl.BlockSpec((_R, 1), lambda i: (i, 0)),
            pl.BlockSpec((128, 16), lambda i: (0, 0)),
        ],
        out_specs=[
            pl.BlockSpec((_R, 128), lambda i: (i, 0)),
            pl.BlockSpec((_R, 16), lambda i: (i, 0)),
        ],
        out_shape=[
            jax.ShapeDtypeStruct((N, 128), jnp.float32),
            jax.ShapeDtypeStruct((N, 16), jnp.float32),
        ],
    )(agg1, nd, b1, ns, w2)


def _tc_out(agg2, nd, b2):
    def body(p_ref, nd_ref, b_ref, h2_ref):
        h2_ref[...] = (p_ref[0] + p_ref[1]) * nd_ref[...] + b_ref[...]

    return pl.pallas_call(
        body,
        grid=(N // _R,),
        in_specs=[
            pl.BlockSpec((NC, _R, 16), lambda i: (0, i, 0)),
            pl.BlockSpec((_R, 1), lambda i: (i, 0)),
            pl.BlockSpec((1, 16), lambda i: (0, 0)),
        ],
        out_specs=pl.BlockSpec((_R, 16), lambda i: (i, 0)),
        out_shape=jax.ShapeDtypeStruct((N, 16), jnp.float32),
    )(agg2, nd, b2)


def kernel(features, edge_index, W1, b1, W2, b2):
    src2d = edge_index[0].reshape(NROW, CH)
    dst2d = edge_index[1].reshape(NROW, CH)
    zeros1 = jnp.zeros((RB,), jnp.float32)
    zd1 = jnp.zeros((CH, 128), jnp.float32)
    zd2 = jnp.zeros((CH, 16), jnp.float32)

    odeg, ideg = _sc_degrees(src2d, dst2d, zeros1)
    hs1, ns, nd = _tc_prep(features, W1,
                           odeg.reshape(NC, N, 1), ideg.reshape(NC, N, 1))
    agg1 = _sc_edge_agg(hs1, src2d, dst2d, zd1, 128).reshape(NC, N, 128)
    h1, g2 = _tc_mid(agg1, nd, b1.reshape(1, 128), ns, W2)
    agg2 = _sc_edge_agg(g2, src2d, dst2d, zd2, 16).reshape(NC, N, 16)
    h2 = _tc_out(agg2, nd, b2.reshape(1, 16))
    return (h2, features, h1, h2)
